# baseline (speedup 1.0000x reference)
import jax
import jax.numpy as jnp
from jax import lax
from jax.experimental import pallas as pl
from jax.experimental.pallas import tpu as pltpu
from jax.experimental.pallas import tpu_sc as plsc

NC = 2
NS = 16
NW = NC * NS

BATCH = 16384
EMBED = 64
PITCH = 2 * EMBED
CROWS = 1000
QROWS = 1000000
BPW = BATCH // NW
BLK = 128
NBLK = BPW // BLK
LANES = 16
KV = EMBED // LANES


def _make_kernel():
  mesh = plsc.VectorSubcoreMesh(core_axis_name="c", subcore_axis_name="s")

  @pl.kernel(
      out_type=jax.ShapeDtypeStruct((BATCH, PITCH), jnp.float32),
      mesh=mesh,
      scratch_types=[
          pltpu.VMEM((2, BPW), jnp.int32),
          pltpu.VMEM((NBLK, BLK), jnp.int32),
          pltpu.VMEM((2, BLK, PITCH), jnp.float32),
          pltpu.VMEM((BPW, PITCH), jnp.float32),
          [pltpu.SemaphoreType.DMA] * NBLK,
          pltpu.SemaphoreType.DMA,
          pltpu.SemaphoreType.DMA,
      ],
  )
  def k(idx_hbm, cidx_hbm, cpk_hbm, qtab_hbm, out_hbm,
        idx_s, cidx_v, bufc, mix, qsems, csem, osem):
    wid = lax.axis_index("s") * NC + lax.axis_index("c")
    base = wid * BPW
    qtab3 = qtab_hbm

    pltpu.sync_copy(idx_hbm.at[wid], idx_s)
    pltpu.sync_copy(cidx_hbm.at[wid], cidx_v)

    def issue_q(b):
      def vec_group(v, _):
        r0 = b * BLK + v * LANES
        qvec = idx_s[1, pl.ds(r0, LANES)]
        for j in range(LANES):
          q = qvec[j]
          pltpu.async_copy(
              qtab3.at[q >> 3, q & 7],
              mix.at[r0 + j, pl.ds(EMBED, EMBED)],
              qsems[b],
          )
        return 0

      lax.fori_loop(0, BLK // LANES, vec_group, 0)

    for b in range(NBLK):
      issue_q(b)

    def stream_color(b):
      pltpu.async_copy(cpk_hbm.at[cidx_v.at[b]], bufc.at[b % 2], csem)

    def assemble_color(b):
      pltpu.make_async_copy(cpk_hbm.at[cidx_v.at[0]], bufc.at[b % 2], csem).wait()

      def vec_group(v, _):
        r0 = b * BLK + v * LANES
        cvec = idx_s[0, pl.ds(r0, LANES)]
        for l in range(LANES):
          ch = (cvec[l] & 1) * EMBED
          for t in range(KV):
            mix[r0 + l, pl.ds(t * LANES, LANES)] = bufc[
                b % 2, v * LANES + l, pl.ds(ch + t * LANES, LANES)
            ]
        return 0

      lax.fori_loop(0, BLK // LANES, vec_group, 0)

    def drain_q(b):
      pltpu.make_async_copy(
          out_hbm.at[pl.ds(0, BLK // 2)],
          mix.at[pl.ds(0, BLK // 2)],
          qsems[b],
      ).wait()

    stream_color(0)
    for b in range(NBLK):
      if b + 1 < NBLK:
        stream_color(b + 1)
      assemble_color(b)
      drain_q(b)
      pltpu.async_copy(
          mix.at[pl.ds(b * BLK, BLK)],
          out_hbm.at[pl.ds(base + b * BLK, BLK)],
          osem,
      )
    for _ in range(NBLK):
      pltpu.make_async_copy(
          mix.at[pl.ds(0, BLK)], out_hbm.at[pl.ds(base, BLK)], osem
      ).wait()

  return k


_kernel = _make_kernel()

RBLK = 8000


def _relin(table):
  def body(t_ref, o_ref):
    o_ref[...] = t_ref[...].reshape(o_ref.shape)

  rows = table.shape[0]
  return pl.pallas_call(
      body,
      grid=(rows // RBLK,),
      in_specs=[pl.BlockSpec((RBLK, EMBED), lambda i: (i, 0))],
      out_specs=pl.BlockSpec((RBLK // 8, 8, EMBED), lambda i: (i, 0, 0)),
      out_shape=jax.ShapeDtypeStruct((rows // 8, 8, EMBED), jnp.float32),
  )(table)


@jax.jit
def kernel(x, color_table, question_table):
  xi = x.astype(jnp.int32).T.reshape(2, NW, BPW).transpose(1, 0, 2)
  cidx = (x[:, 0].astype(jnp.int32) >> 1).reshape(NW, NBLK, BLK)
  cpk = color_table.reshape(CROWS // 2, PITCH)
  qtab3 = _relin(question_table)
  return _kernel(xi, cidx, cpk, qtab3)


# speedup vs baseline: 2.7502x; 2.7502x over previous
import jax
import jax.numpy as jnp
from jax import lax
from jax.experimental import pallas as pl
from jax.experimental.pallas import tpu as pltpu
from jax.experimental.pallas import tpu_sc as plsc

NC = 2
NS = 16
NW = NC * NS

BATCH = 16384
EMBED = 64
PITCH = 2 * EMBED
CROWS = 1000
QROWS = 1000000
BPW = BATCH // NW
BLK = 128
NBLK = BPW // BLK
LANES = 16
KV = EMBED // LANES


def _make_kernel():
  mesh = plsc.VectorSubcoreMesh(core_axis_name="c", subcore_axis_name="s")

  @pl.kernel(
      out_type=jax.ShapeDtypeStruct((BATCH, PITCH), jnp.float32),
      mesh=mesh,
      scratch_types=[
          pltpu.VMEM((2, BPW), jnp.int32),
          pltpu.VMEM((NBLK, BLK), jnp.int32),
          pltpu.VMEM((2, BLK, PITCH), jnp.float32),
          pltpu.VMEM((BPW, PITCH), jnp.float32),
          [pltpu.SemaphoreType.DMA] * NBLK,
          pltpu.SemaphoreType.DMA,
          pltpu.SemaphoreType.DMA,
      ],
  )
  def k(idx_hbm, cidx_hbm, cpk_hbm, qtab_hbm, out_hbm,
        idx_s, cidx_v, bufc, mix, qsems, csem, osem):
    wid = lax.axis_index("s") * NC + lax.axis_index("c")
    base = wid * BPW
    qtab3 = qtab_hbm

    pltpu.sync_copy(idx_hbm.at[wid], idx_s)
    pltpu.sync_copy(cidx_hbm.at[wid], cidx_v)

    def issue_q(b):
      def vec_group(v, _):
        r0 = b * BLK + v * LANES
        qvec = idx_s[1, pl.ds(r0, LANES)]
        for j in range(LANES):
          q = qvec[j]
          pltpu.async_copy(
              qtab3.at[q >> 3, q & 7],
              mix.at[r0 + j, pl.ds(EMBED, EMBED)],
              qsems[b],
          )
        return 0

      lax.fori_loop(0, BLK // LANES, vec_group, 0)

    for b in range(NBLK):
      issue_q(b)

    def stream_color(b):
      pltpu.async_copy(cpk_hbm.at[cidx_v.at[b]], bufc.at[b % 2], csem)

    def assemble_color(b):
      pltpu.make_async_copy(cpk_hbm.at[cidx_v.at[0]], bufc.at[b % 2], csem).wait()

      def vec_group(v, _):
        r0 = b * BLK + v * LANES
        cvec = idx_s[0, pl.ds(r0, LANES)]
        for l in range(LANES):
          ch = (cvec[l] & 1) * EMBED
          for t in range(KV):
            mix[r0 + l, pl.ds(t * LANES, LANES)] = bufc[
                b % 2, v * LANES + l, pl.ds(ch + t * LANES, LANES)
            ]
        return 0

      lax.fori_loop(0, BLK // LANES, vec_group, 0)

    def drain_q(b):
      pltpu.make_async_copy(
          out_hbm.at[pl.ds(0, BLK // 2)],
          mix.at[pl.ds(0, BLK // 2)],
          qsems[b],
      ).wait()

    stream_color(0)
    for b in range(NBLK):
      if b + 1 < NBLK:
        stream_color(b + 1)
      assemble_color(b)
      drain_q(b)
      pltpu.async_copy(
          mix.at[pl.ds(b * BLK, BLK)],
          out_hbm.at[pl.ds(base + b * BLK, BLK)],
          osem,
      )
    for _ in range(NBLK):
      pltpu.make_async_copy(
          mix.at[pl.ds(0, BLK)], out_hbm.at[pl.ds(base, BLK)], osem
      ).wait()

  return k


_kernel = _make_kernel()


@jax.jit
def kernel(x, color_table, question_table):
  xi = x.astype(jnp.int32).T.reshape(2, NW, BPW).transpose(1, 0, 2)
  cidx = (x[:, 0].astype(jnp.int32) >> 1).reshape(NW, NBLK, BLK)
  cpk = color_table.reshape(CROWS // 2, PITCH)
  qtab3 = question_table.reshape(QROWS // 8, 8, EMBED)
  return _kernel(xi, cidx, cpk, qtab3)


# speedup vs baseline: 2.7552x; 1.0018x over previous
import jax
import jax.numpy as jnp
from jax import lax
from jax.experimental import pallas as pl
from jax.experimental.pallas import tpu as pltpu
from jax.experimental.pallas import tpu_sc as plsc

NC = 2
NS = 16
NW = NC * NS

BATCH = 16384
EMBED = 64
PITCH = 2 * EMBED
CROWS = 1000
QROWS = 1000000
BPW = BATCH // NW
BLK = 128
NBLK = BPW // BLK
LANES = 16
KV = EMBED // LANES


def _make_kernel():
  mesh = plsc.VectorSubcoreMesh(core_axis_name="c", subcore_axis_name="s")

  @pl.kernel(
      out_type=jax.ShapeDtypeStruct((BATCH, PITCH), jnp.float32),
      mesh=mesh,
      scratch_types=[
          pltpu.VMEM((2, BPW), jnp.int32),
          pltpu.VMEM((NBLK, BLK), jnp.int32),
          pltpu.VMEM((2, BLK, PITCH), jnp.float32),
          pltpu.VMEM((BPW, PITCH), jnp.float32),
          [pltpu.SemaphoreType.DMA] * NBLK,
          pltpu.SemaphoreType.DMA,
          pltpu.SemaphoreType.DMA,
      ],
  )
  def k(idx_hbm, cidx_hbm, cpk_hbm, qtab3, out_hbm,
        idx_s, cidx_v, bufc, mix, qsems, csem, osem):
    wid = lax.axis_index("s") * NC + lax.axis_index("c")
    base = wid * BPW

    pltpu.sync_copy(idx_hbm.at[wid], idx_s)
    pltpu.sync_copy(cidx_hbm.at[wid], cidx_v)

    def issue_q(b):
      def vec_group(v, _):
        r0 = b * BLK + v * LANES
        qvec = idx_s[1, pl.ds(r0, LANES)]
        for j in range(LANES):
          q = qvec[j]
          pltpu.async_copy(
              qtab3.at[q >> 3, q & 7],
              mix.at[r0 + j, pl.ds(EMBED, EMBED)],
              qsems[b],
          )
        return 0

      lax.fori_loop(0, BLK // LANES, vec_group, 0)

    for b in range(NBLK):
      issue_q(b)

    def stream_color(b):
      pltpu.async_copy(cpk_hbm.at[cidx_v.at[b]], bufc.at[b % 2], csem)

    def assemble_color(b):
      pltpu.make_async_copy(cpk_hbm.at[cidx_v.at[0]], bufc.at[b % 2], csem).wait()

      def vec_group(v, _):
        r0 = b * BLK + v * LANES
        cvec = idx_s[0, pl.ds(r0, LANES)]
        for l in range(LANES):
          ch = (cvec[l] & 1) * EMBED
          for t in range(KV):
            mix[r0 + l, pl.ds(t * LANES, LANES)] = bufc[
                b % 2, v * LANES + l, pl.ds(ch + t * LANES, LANES)
            ]
        return 0

      lax.fori_loop(0, BLK // LANES, vec_group, 0)

    def drain_q(b):
      pltpu.make_async_copy(
          out_hbm.at[pl.ds(0, BLK // 2)],
          mix.at[pl.ds(0, BLK // 2)],
          qsems[b],
      ).wait()

    stream_color(0)
    for b in range(NBLK):
      if b + 1 < NBLK:
        stream_color(b + 1)
      assemble_color(b)
      drain_q(b)
      pltpu.async_copy(
          mix.at[pl.ds(b * BLK, BLK)],
          out_hbm.at[pl.ds(base + b * BLK, BLK)],
          osem,
      )
    for _ in range(NBLK):
      pltpu.make_async_copy(
          mix.at[pl.ds(0, BLK)], out_hbm.at[pl.ds(base, BLK)], osem
      ).wait()

  return k


_kernel = _make_kernel()


@jax.jit
def kernel(x, color_table, question_table):
  xi = x.astype(jnp.int32).T.reshape(2, NW, BPW).transpose(1, 0, 2)
  cidx = (x[:, 0].astype(jnp.int32) >> 1).reshape(NW, NBLK, BLK)
  cpk = color_table.reshape(CROWS // 2, PITCH)
  qtab3 = question_table.reshape(QROWS // 8, 8, EMBED)
  return _kernel(xi, cidx, cpk, qtab3)
